# baseline (speedup 1.0000x reference)
import functools

import jax
import jax.numpy as jnp
from jax import lax
from jax.experimental import pallas as pl
from jax.experimental.pallas import tpu as pltpu
from jax.experimental.pallas import tpu_sc as plsc

NC, NS = 2, 16
NW = NC * NS
LANE = 128
CW = 1408


def _make_sc_gather(b, v, d):
    rpw = b // NW
    dal = (d // LANE) * LANE
    nb = dal // CW
    assert b % NW == 0 and rpw % 8 == 0 and dal % CW == 0 and nb >= 3

    mesh = plsc.VectorSubcoreMesh(core_axis_name="c", subcore_axis_name="s")

    @functools.partial(
        pl.kernel,
        out_type=jax.ShapeDtypeStruct((b, 1, d), jnp.float32),
        mesh=mesh,
        scratch_types=[
            pltpu.VMEM((rpw,), jnp.int32),
            pltpu.VMEM((2, rpw, 1, CW), jnp.float32),
            pltpu.SemaphoreType.DMA((2,)),
            pltpu.SemaphoreType.DMA((2,)),
        ],
    )
    def gather(e_hbm, table_hbm, out_hbm, idx_v, bufs, gsem, ssem):
        wid = lax.axis_index("s") * NC + lax.axis_index("c")
        base = wid * rpw
        pltpu.sync_copy(e_hbm.at[pl.ds(base, rpw)], idx_v)

        def gcopy(c, slot):
            return pltpu.make_async_copy(
                table_hbm.at[idx_v, pl.ds(c * CW, CW)],
                bufs.at[slot, :, 0],
                gsem.at[slot],
            )

        def scopy(c, slot):
            return pltpu.make_async_copy(
                bufs.at[slot],
                out_hbm.at[pl.ds(base, rpw), pl.ds(0, 1), pl.ds(c * CW, CW)],
                ssem.at[slot],
            )

        gcopy(0, 0).start()
        gcopy(1, 1).start()
        gcopy(0, 0).wait()
        scopy(0, 0).start()

        def body(c, _):
            slot = lax.rem(c, 2)
            nslot = lax.rem(c + 1, 2)
            gcopy(c, slot).wait()
            scopy(c, slot).start()
            scopy(c - 1, nslot).wait()
            gcopy(c + 1, nslot).start()
            return ()

        lax.fori_loop(1, nb - 1, body, ())

        last = nb - 1
        lslot = lax.rem(last, 2)
        gcopy(last, lslot).wait()
        scopy(last, lslot).start()
        scopy(last - 1, lax.rem(last + 1, 2)).wait()
        scopy(last, lslot).wait()

    return gather


def _make_tc_tail(b, v, d):
    dal = (d // LANE) * LANE
    tailblk = dal // LANE

    def tail_kernel(e_ref, wtail_ref, _, o_ref):
        e = e_ref[:]
        onehot = (
            e[:, None] == lax.broadcasted_iota(jnp.int32, (b, v), 1)
        ).astype(jnp.float32)
        res = jnp.dot(onehot, wtail_ref[...], preferred_element_type=jnp.float32)
        o_ref[...] = res[:, None, :]

    return pl.pallas_call(
        tail_kernel,
        grid=(1,),
        out_shape=jax.ShapeDtypeStruct((b, 1, d), jnp.float32),
        in_specs=[
            pl.BlockSpec((b,), lambda i: (0,)),
            pl.BlockSpec((v, LANE), lambda i: (0, tailblk)),
            pl.BlockSpec(memory_space=pl.ANY),
        ],
        out_specs=pl.BlockSpec((b, 1, LANE), lambda i: (0, 0, tailblk)),
        input_output_aliases={2: 0},
    )


def kernel(e, weight):
    b = e.shape[0]
    v, d = weight.shape
    ei = e.astype(jnp.int32)
    out = _make_sc_gather(b, v, d)(ei, weight)
    return _make_tc_tail(b, v, d)(ei, weight, out)
